# baseline (speedup 1.0000x reference)
import functools

import jax
import jax.numpy as jnp
from jax import lax
from jax.experimental import pallas as pl
from jax.experimental.pallas import tpu as pltpu
from jax.experimental.pallas import tpu_sc as plsc

N = 10000
E = 320000
D = 128
H = 128
B = 100
NPG = 100

NC = 2
NS = 16
NW = NC * NS
EPW = E // NW
CHUNK = 80
NCHUNK = EPW // CHUNK
ROWS_A = 632
LAST_A = N - (NS - 1) * ROWS_A
NB = 2


def _sc_segment_sum(h, src3, dst3, zeros):
    mesh = plsc.VectorSubcoreMesh(core_axis_name="c", subcore_axis_name="s")

    def body(h_hbm, src_hbm, dst_hbm, zeros_hbm, out_hbm,
             srcb, d0, d1, r0, r1, acc,
             ms0, md0, md1, m0, m1):
        dstb = (d0, d1)
        rows = (r0, r1)
        sems = (m0, m1)
        sem_d = (md0, md1)
        c = lax.axis_index("c")
        s = lax.axis_index("s")
        wid = s * NC + c

        pltpu.async_copy(src_hbm.at[wid], srcb, ms0)
        for q in range(NB):
            pltpu.async_copy(dst_hbm.at[wid, q], dstb[q], sem_d[q])

        @pl.when(s < NS - 1)
        def _():
            pltpu.sync_copy(zeros_hbm.at[pl.ds(s * ROWS_A, ROWS_A)],
                            acc.at[pl.ds(s * ROWS_A, ROWS_A)])

        @pl.when(s == NS - 1)
        def _():
            pltpu.sync_copy(zeros_hbm.at[pl.ds((NS - 1) * ROWS_A, LAST_A)],
                            acc.at[pl.ds((NS - 1) * ROWS_A, LAST_A)])

        pltpu.make_async_copy(src_hbm.at[wid], srcb, ms0).wait()
        plsc.subcore_barrier()

        for q in range(NB):
            pltpu.async_copy(h_hbm.at[srcb.at[q]], rows[q], sems[q])

        def blk(b, carry):
            for j in range(NB):
                t = b * NB + j
                pltpu.make_async_copy(h_hbm.at[srcb.at[0]],
                                      rows[j], sems[j]).wait()
                pltpu.make_async_copy(dst_hbm.at[wid, 0], dstb[j],
                                      sem_d[j]).wait()
                pltpu.sync_copy(rows[j], acc.at[dstb[j]], add=True)

                @pl.when(t + NB < NCHUNK)
                def _():
                    pltpu.async_copy(dst_hbm.at[wid, t + NB], dstb[j],
                                     sem_d[j])
                    pltpu.async_copy(h_hbm.at[srcb.at[t + NB]], rows[j],
                                     sems[j])
            return carry

        lax.fori_loop(0, NCHUNK // NB, blk, 0)
        tj = (NCHUNK - 1) % NB
        pltpu.make_async_copy(h_hbm.at[srcb.at[0]], rows[tj],
                              sems[tj]).wait()
        pltpu.make_async_copy(dst_hbm.at[wid, 0], dstb[tj],
                              sem_d[tj]).wait()
        pltpu.sync_copy(rows[tj], acc.at[dstb[tj]], add=True)

        plsc.subcore_barrier()

        @pl.when(s < NS - 1)
        def _():
            pltpu.sync_copy(acc.at[pl.ds(s * ROWS_A, ROWS_A)],
                            out_hbm.at[c, pl.ds(s * ROWS_A, ROWS_A)])

        @pl.when(s == NS - 1)
        def _():
            pltpu.sync_copy(acc.at[pl.ds((NS - 1) * ROWS_A, LAST_A)],
                            out_hbm.at[c, pl.ds((NS - 1) * ROWS_A, LAST_A)])

    return pl.kernel(
        body,
        out_type=jax.ShapeDtypeStruct((NC, N, D), jnp.float32),
        mesh=mesh,
        scratch_types=(
            [pltpu.VMEM((NCHUNK, CHUNK), jnp.int32)]
            + [pltpu.VMEM((CHUNK,), jnp.int32) for _ in range(NB)]
            + [pltpu.VMEM((CHUNK, D), jnp.float32) for _ in range(NB)]
            + [pltpu.VMEM_SHARED((N, D), jnp.float32)]
            + [pltpu.SemaphoreType.DMA for _ in range(1 + 2 * NB)]
        ),
    )(h, src3, dst3, zeros)


RB = 2000


def _mlp1(aggp, x, W1a, b1a, W1b, b1b):
    def body(a0, a1, xr, wa, ba, wb, bb, o):
        t = a0[0] + a1[0] + xr[...]
        u = jnp.maximum(jnp.dot(t, wa[...],
                                preferred_element_type=jnp.float32) + ba[...], 0.0)
        o[...] = jnp.maximum(jnp.dot(u, wb[...],
                                     preferred_element_type=jnp.float32) + bb[...], 0.0)

    grid = N // RB
    return pl.pallas_call(
        body,
        grid=(grid,),
        in_specs=[
            pl.BlockSpec((1, RB, D), lambda i: (0, i, 0)),
            pl.BlockSpec((1, RB, D), lambda i: (1, i, 0)),
            pl.BlockSpec((RB, D), lambda i: (i, 0)),
            pl.BlockSpec((D, H), lambda i: (0, 0)),
            pl.BlockSpec((1, H), lambda i: (0, 0)),
            pl.BlockSpec((H, H), lambda i: (0, 0)),
            pl.BlockSpec((1, H), lambda i: (0, 0)),
        ],
        out_specs=pl.BlockSpec((RB, H), lambda i: (i, 0)),
        out_shape=jax.ShapeDtypeStruct((N, H), jnp.float32),
    )(aggp, aggp, x, W1a, b1a.reshape(1, H), W1b, b1b.reshape(1, H))


def _mlp2_actor(aggp, h1, W2a, b2a, W2b, b2b, A1, ba1, A2, ba2, A3, ba3):
    def body(a0, a1, hr, wa, ba, wb, bb, a1w, ab1, a2w, ab2, a3w, ab3,
             emb_o, log_o):
        t = a0[0] + a1[0] + hr[...]
        u = jnp.maximum(jnp.dot(t, wa[...],
                                preferred_element_type=jnp.float32) + ba[...], 0.0)
        e = jnp.maximum(jnp.dot(u, wb[...],
                                preferred_element_type=jnp.float32) + bb[...], 0.0)
        emb_o[...] = e
        a = jnp.tanh(jnp.dot(e, a1w[...],
                             preferred_element_type=jnp.float32) + ab1[...])
        a = jnp.tanh(jnp.dot(a, a2w[...],
                             preferred_element_type=jnp.float32) + ab2[...])
        log_o[...] = jnp.dot(a, a3w[...],
                             preferred_element_type=jnp.float32) + ab3[...]

    grid = N // RB
    full = lambda i: (0, 0)
    return pl.pallas_call(
        body,
        grid=(grid,),
        in_specs=[
            pl.BlockSpec((1, RB, D), lambda i: (0, i, 0)),
            pl.BlockSpec((1, RB, D), lambda i: (1, i, 0)),
            pl.BlockSpec((RB, D), lambda i: (i, 0)),
            pl.BlockSpec((H, H), full),
            pl.BlockSpec((1, H), full),
            pl.BlockSpec((H, H), full),
            pl.BlockSpec((1, H), full),
            pl.BlockSpec((H, H), full),
            pl.BlockSpec((1, H), full),
            pl.BlockSpec((H, H), full),
            pl.BlockSpec((1, H), full),
            pl.BlockSpec((H, 1), full),
            pl.BlockSpec((1, 1), full),
        ],
        out_specs=[
            pl.BlockSpec((RB, H), lambda i: (i, 0)),
            pl.BlockSpec((RB, 1), lambda i: (i, 0)),
        ],
        out_shape=[
            jax.ShapeDtypeStruct((N, H), jnp.float32),
            jax.ShapeDtypeStruct((N, 1), jnp.float32),
        ],
    )(aggp, aggp, h1, W2a, b2a.reshape(1, H), W2b, b2b.reshape(1, H),
      A1, ba1.reshape(1, H), A2, ba2.reshape(1, H), A3, ba3.reshape(1, 1))


def _decode(logits, mask):
    def body(l_ref, m_ref, lp_ref, act_ref):
        l = l_ref[...]
        mx = jnp.max(l, axis=1, keepdims=True)
        lse = jnp.log(jnp.sum(jnp.exp(l - mx), axis=1, keepdims=True)) + mx
        lp = l - lse
        lp_ref[...] = lp
        probs = jnp.exp(lp)
        masked = jnp.where(m_ref[...] > 0, probs, -jnp.inf)
        rmax = jnp.max(masked, axis=1, keepdims=True)
        idx = lax.broadcasted_iota(jnp.int32, (B, NPG), 1)
        cand = jnp.where(masked == rmax, idx, NPG)
        act_ref[...] = jnp.min(cand, axis=1, keepdims=True)

    lp, act = pl.pallas_call(
        body,
        out_shape=[
            jax.ShapeDtypeStruct((B, NPG), jnp.float32),
            jax.ShapeDtypeStruct((B, 1), jnp.int32),
        ],
    )(logits, mask)
    return lp, act[:, 0]


def kernel(x, edge_index, mask, W1a, b1a, W1b, b1b, W2a, b2a, W2b, b2b,
           A1, ba1, A2, ba2, A3, ba3):
    src3 = edge_index[0].reshape(NW, NCHUNK, CHUNK)
    dst3 = edge_index[1].reshape(NW, NCHUNK, CHUNK)
    zeros = jnp.zeros((N, D), jnp.float32)
    aggp1 = _sc_segment_sum(x, src3, dst3, zeros)
    h1 = _mlp1(aggp1, x, W1a, b1a, W1b, b1b)
    aggp2 = _sc_segment_sum(h1, src3, dst3, zeros)
    emb, logits = _mlp2_actor(aggp2, h1, W2a, b2a, W2b, b2b,
                              A1, ba1, A2, ba2, A3, ba3)
    log_p, actions = _decode(logits.reshape(B, NPG), mask)
    return (log_p, actions, emb)


# speedup vs baseline: 1.1952x; 1.1952x over previous
import functools

import jax
import jax.numpy as jnp
from jax import lax
from jax.experimental import pallas as pl
from jax.experimental.pallas import tpu as pltpu
from jax.experimental.pallas import tpu_sc as plsc

N = 10000
E = 320000
D = 128
H = 128
B = 100
NPG = 100

NC = 2
NS = 16
NW = NC * NS
EPW = E // NW
CHUNK = 40
NCHUNK = EPW // CHUNK
ROWS_A = 632
LAST_A = N - (NS - 1) * ROWS_A
NB = 5


def _sc_segment_sum(h, src3, dst3, zeros):
    mesh = plsc.VectorSubcoreMesh(core_axis_name="c", subcore_axis_name="s")

    def body(h_hbm, src_hbm, dst_hbm, zeros_hbm, out_hbm,
             srcb, d0, d1, d2, d3, d4, r0, r1, r2, r3, r4, acc,
             ms0, md0, md1, md2, md3, md4, m0, m1, m2, m3, m4):
        dstb = (d0, d1, d2, d3, d4)
        rows = (r0, r1, r2, r3, r4)
        sems = (m0, m1, m2, m3, m4)
        sem_d = (md0, md1, md2, md3, md4)
        c = lax.axis_index("c")
        s = lax.axis_index("s")
        wid = s * NC + c

        pltpu.async_copy(src_hbm.at[wid], srcb, ms0)
        for q in range(NB):
            pltpu.async_copy(dst_hbm.at[wid, q], dstb[q], sem_d[q])

        @pl.when(s < NS - 1)
        def _():
            pltpu.sync_copy(zeros_hbm.at[pl.ds(s * ROWS_A, ROWS_A)],
                            acc.at[pl.ds(s * ROWS_A, ROWS_A)])

        @pl.when(s == NS - 1)
        def _():
            pltpu.sync_copy(zeros_hbm.at[pl.ds((NS - 1) * ROWS_A, LAST_A)],
                            acc.at[pl.ds((NS - 1) * ROWS_A, LAST_A)])

        pltpu.make_async_copy(src_hbm.at[wid], srcb, ms0).wait()
        plsc.subcore_barrier()

        for q in range(NB):
            pltpu.async_copy(h_hbm.at[srcb.at[pl.ds(q * CHUNK, CHUNK)]],
                             rows[q], sems[q])

        def blk(b, carry):
            for j in range(NB):
                t = b * NB + j
                pltpu.make_async_copy(h_hbm.at[srcb.at[pl.ds(0, CHUNK)]],
                                      rows[j], sems[j]).wait()
                pltpu.make_async_copy(dst_hbm.at[wid, 0], dstb[j],
                                      sem_d[j]).wait()
                pltpu.sync_copy(rows[j], acc.at[dstb[j]], add=True)

                @pl.when(t + NB < NCHUNK)
                def _():
                    pltpu.async_copy(dst_hbm.at[wid, t + NB], dstb[j],
                                     sem_d[j])
                    pltpu.async_copy(
                        h_hbm.at[srcb.at[pl.ds((t + NB) * CHUNK, CHUNK)]],
                        rows[j], sems[j])
            return carry

        lax.fori_loop(0, NCHUNK // NB, blk, 0)
        plsc.subcore_barrier()

        @pl.when(s < NS - 1)
        def _():
            pltpu.sync_copy(acc.at[pl.ds(s * ROWS_A, ROWS_A)],
                            out_hbm.at[c, pl.ds(s * ROWS_A, ROWS_A)])

        @pl.when(s == NS - 1)
        def _():
            pltpu.sync_copy(acc.at[pl.ds((NS - 1) * ROWS_A, LAST_A)],
                            out_hbm.at[c, pl.ds((NS - 1) * ROWS_A, LAST_A)])

    return pl.kernel(
        body,
        out_type=jax.ShapeDtypeStruct((NC, N, D), jnp.float32),
        mesh=mesh,
        scratch_types=(
            [pltpu.VMEM((EPW,), jnp.int32)]
            + [pltpu.VMEM((CHUNK,), jnp.int32) for _ in range(NB)]
            + [pltpu.VMEM((CHUNK, D), jnp.float32) for _ in range(NB)]
            + [pltpu.VMEM_SHARED((N, D), jnp.float32)]
            + [pltpu.SemaphoreType.DMA for _ in range(1 + 2 * NB)]
        ),
    )(h, src3, dst3, zeros)


RB = 2000


def _mlp1(aggp, x, W1a, b1a, W1b, b1b):
    def body(a0, a1, xr, wa, ba, wb, bb, o):
        t = a0[0] + a1[0] + xr[...]
        u = jnp.maximum(jnp.dot(t, wa[...],
                                preferred_element_type=jnp.float32) + ba[...], 0.0)
        o[...] = jnp.maximum(jnp.dot(u, wb[...],
                                     preferred_element_type=jnp.float32) + bb[...], 0.0)

    grid = N // RB
    return pl.pallas_call(
        body,
        grid=(grid,),
        in_specs=[
            pl.BlockSpec((1, RB, D), lambda i: (0, i, 0)),
            pl.BlockSpec((1, RB, D), lambda i: (1, i, 0)),
            pl.BlockSpec((RB, D), lambda i: (i, 0)),
            pl.BlockSpec((D, H), lambda i: (0, 0)),
            pl.BlockSpec((1, H), lambda i: (0, 0)),
            pl.BlockSpec((H, H), lambda i: (0, 0)),
            pl.BlockSpec((1, H), lambda i: (0, 0)),
        ],
        out_specs=pl.BlockSpec((RB, H), lambda i: (i, 0)),
        out_shape=jax.ShapeDtypeStruct((N, H), jnp.float32),
    )(aggp, aggp, x, W1a, b1a.reshape(1, H), W1b, b1b.reshape(1, H))


def _mlp2_actor(aggp, h1, W2a, b2a, W2b, b2b, A1, ba1, A2, ba2, A3, ba3):
    def body(a0, a1, hr, wa, ba, wb, bb, a1w, ab1, a2w, ab2, a3w, ab3,
             emb_o, log_o):
        t = a0[0] + a1[0] + hr[...]
        u = jnp.maximum(jnp.dot(t, wa[...],
                                preferred_element_type=jnp.float32) + ba[...], 0.0)
        e = jnp.maximum(jnp.dot(u, wb[...],
                                preferred_element_type=jnp.float32) + bb[...], 0.0)
        emb_o[...] = e
        a = jnp.tanh(jnp.dot(e, a1w[...],
                             preferred_element_type=jnp.float32) + ab1[...])
        a = jnp.tanh(jnp.dot(a, a2w[...],
                             preferred_element_type=jnp.float32) + ab2[...])
        log_o[...] = jnp.dot(a, a3w[...],
                             preferred_element_type=jnp.float32) + ab3[...]

    grid = N // RB
    full = lambda i: (0, 0)
    return pl.pallas_call(
        body,
        grid=(grid,),
        in_specs=[
            pl.BlockSpec((1, RB, D), lambda i: (0, i, 0)),
            pl.BlockSpec((1, RB, D), lambda i: (1, i, 0)),
            pl.BlockSpec((RB, D), lambda i: (i, 0)),
            pl.BlockSpec((H, H), full),
            pl.BlockSpec((1, H), full),
            pl.BlockSpec((H, H), full),
            pl.BlockSpec((1, H), full),
            pl.BlockSpec((H, H), full),
            pl.BlockSpec((1, H), full),
            pl.BlockSpec((H, H), full),
            pl.BlockSpec((1, H), full),
            pl.BlockSpec((H, 1), full),
            pl.BlockSpec((1, 1), full),
        ],
        out_specs=[
            pl.BlockSpec((RB, H), lambda i: (i, 0)),
            pl.BlockSpec((RB, 1), lambda i: (i, 0)),
        ],
        out_shape=[
            jax.ShapeDtypeStruct((N, H), jnp.float32),
            jax.ShapeDtypeStruct((N, 1), jnp.float32),
        ],
    )(aggp, aggp, h1, W2a, b2a.reshape(1, H), W2b, b2b.reshape(1, H),
      A1, ba1.reshape(1, H), A2, ba2.reshape(1, H), A3, ba3.reshape(1, 1))


def _decode(logits, mask):
    def body(l_ref, m_ref, lp_ref, act_ref):
        l = l_ref[...]
        mx = jnp.max(l, axis=1, keepdims=True)
        lse = jnp.log(jnp.sum(jnp.exp(l - mx), axis=1, keepdims=True)) + mx
        lp = l - lse
        lp_ref[...] = lp
        probs = jnp.exp(lp)
        masked = jnp.where(m_ref[...] > 0, probs, -jnp.inf)
        rmax = jnp.max(masked, axis=1, keepdims=True)
        idx = lax.broadcasted_iota(jnp.int32, (B, NPG), 1)
        cand = jnp.where(masked == rmax, idx, NPG)
        act_ref[...] = jnp.min(cand, axis=1, keepdims=True)

    lp, act = pl.pallas_call(
        body,
        out_shape=[
            jax.ShapeDtypeStruct((B, NPG), jnp.float32),
            jax.ShapeDtypeStruct((B, 1), jnp.int32),
        ],
    )(logits, mask)
    return lp, act[:, 0]


def kernel(x, edge_index, mask, W1a, b1a, W1b, b1b, W2a, b2a, W2b, b2b,
           A1, ba1, A2, ba2, A3, ba3):
    src3 = edge_index[0].reshape(NW, EPW)
    dst3 = edge_index[1].reshape(NW, NCHUNK, CHUNK)
    zeros = jnp.zeros((N, D), jnp.float32)
    aggp1 = _sc_segment_sum(x, src3, dst3, zeros)
    h1 = _mlp1(aggp1, x, W1a, b1a, W1b, b1b)
    aggp2 = _sc_segment_sum(h1, src3, dst3, zeros)
    emb, logits = _mlp2_actor(aggp2, h1, W2a, b2a, W2b, b2b,
                              A1, ba1, A2, ba2, A3, ba3)
    log_p, actions = _decode(logits.reshape(B, NPG), mask)
    return (log_p, actions, emb)
